# baseline (speedup 1.0000x reference)
import functools

import jax
import jax.numpy as jnp
from jax import lax
from jax.experimental import pallas as pl
from jax.experimental.pallas import tpu as pltpu
from jax.experimental.pallas import tpu_sc as plsc

N = 10000
NP = 10112
E = 320000
D = 128
NC = 2
NS = 16
NW = NC * NS
CH = 128
NCH = 80
G = 4
NG = NCH // G
NIT = NCH // 8
EPW = CH * NCH
EW = E // NW
RPT = NP // NS

_mesh = plsc.VectorSubcoreMesh(core_axis_name="c", subcore_axis_name="s")


@functools.partial(
    pl.kernel,
    mesh=_mesh,
    out_type=jax.ShapeDtypeStruct((NC, NP, D), jnp.float32),
    scratch_types=[
        pltpu.VMEM((NCH, CH), jnp.int32),
        pltpu.VMEM((CH, D), jnp.float32),
        pltpu.VMEM_SHARED((NP, D), jnp.float32),
        pltpu.SemaphoreType.DMA,
    ],
)
def _sc_deg(dst_hbm, z128_hbm, ones_hbm,
            d_out,
            dst_v, ones_v, deg_sh, sem):
    c = lax.axis_index("c")
    s = lax.axis_index("s")
    w = c * NS + s

    pltpu.sync_copy(dst_hbm.at[w], dst_v)
    pltpu.sync_copy(z128_hbm.at[pl.ds(s * RPT, RPT)], deg_sh.at[pl.ds(s * RPT, RPT)])
    pltpu.sync_copy(ones_hbm, ones_v)

    plsc.subcore_barrier()

    K = 4

    def _chunk(j, carry):
        pltpu.async_copy(ones_v, deg_sh.at[dst_v.at[j]], sem, add=True)

        @pl.when(j >= K)
        def _():
            pltpu.make_async_copy(ones_v, deg_sh.at[dst_v.at[0]], sem).wait()
        return carry
    lax.fori_loop(0, NCH, _chunk, 0)
    for _ in range(K):
        pltpu.make_async_copy(ones_v, deg_sh.at[dst_v.at[0]], sem).wait()

    plsc.subcore_barrier()

    pltpu.sync_copy(deg_sh.at[pl.ds(s * RPT, RPT)], d_out.at[c, pl.ds(s * RPT, RPT)])


@functools.partial(
    pl.kernel,
    mesh=_mesh,
    out_type=jax.ShapeDtypeStruct((NC, NP, D), jnp.float32),
    scratch_types=[
        pltpu.VMEM((NCH, CH), jnp.int32),
        pltpu.VMEM((G, CH), jnp.int32),
        pltpu.VMEM((G, CH), jnp.int32),
        pltpu.VMEM((CH, D), jnp.float32),
        pltpu.VMEM((CH, D), jnp.float32),
        pltpu.VMEM_SHARED((NP, D), jnp.float32),
        pltpu.SemaphoreType.DMA,
        pltpu.SemaphoreType.DMA,
        pltpu.SemaphoreType.DMA,
        pltpu.SemaphoreType.DMA,
        pltpu.SemaphoreType.DMA,
        pltpu.SemaphoreType.DMA,
    ],
)
def _sc_agg(x_hbm, src_hbm, dstg_hbm, z128_hbm,
            p_out,
            src_v, dstg_a, dstg_b, rows_a, rows_b, acc_sh,
            sem_ga, sem_gb, sem_da, sem_db, sem_sa, sem_sb):
    c = lax.axis_index("c")
    s = lax.axis_index("s")
    w = c * NS + s

    pltpu.sync_copy(src_hbm.at[w], src_v)
    pltpu.async_copy(dstg_hbm.at[w, 0], dstg_a, sem_da)
    pltpu.sync_copy(z128_hbm.at[pl.ds(s * RPT, RPT)], acc_sh.at[pl.ds(s * RPT, RPT)])

    plsc.subcore_barrier()

    rows = (rows_a, rows_b)
    sems = (sem_ga, sem_gb)
    ssems = (sem_sa, sem_sb)

    pltpu.async_copy(x_hbm.at[src_v.at[0]], rows_a, sem_ga)

    def _iter(i, carry):
        k0 = 8 * i
        pltpu.make_async_copy(dstg_hbm.at[w, 0], dstg_a, sem_da).wait()
        for t in range(8):
            k = k0 + t
            rb, sg, ss = rows[t % 2], sems[t % 2], ssems[t % 2]
            nrb, nsg, nss = rows[(t + 1) % 2], sems[(t + 1) % 2], ssems[(t + 1) % 2]
            pltpu.make_async_copy(x_hbm.at[src_v.at[k]], rb, sg).wait()
            if t == 0:
                @pl.when(i > 0)
                def _():
                    pltpu.make_async_copy(rows_b, acc_sh.at[dstg_a.at[0]], nss).wait()
                pltpu.async_copy(dstg_hbm.at[w, 2 * i + 1], dstg_b, sem_db)
            else:
                pltpu.make_async_copy(nrb, acc_sh.at[dstg_a.at[0]], nss).wait()
            if t < 7:
                pltpu.async_copy(x_hbm.at[src_v.at[k + 1]], nrb, nsg)
            else:
                @pl.when(i < NIT - 1)
                def _():
                    pltpu.async_copy(x_hbm.at[src_v.at[k + 1]], nrb, nsg)
            if t == 4:
                pltpu.make_async_copy(dstg_hbm.at[w, 0], dstg_b, sem_db).wait()
            dref = dstg_a.at[t] if t < 4 else dstg_b.at[t - 4]
            pltpu.async_copy(rb, acc_sh.at[dref], sem=ss, add=True)

        @pl.when(i < NIT - 1)
        def _():
            pltpu.async_copy(dstg_hbm.at[w, 2 * i + 2], dstg_a, sem_da)
        return carry
    lax.fori_loop(0, NIT, _iter, 0)

    pltpu.make_async_copy(rows_b, acc_sh.at[dstg_a.at[0]], sem_sb).wait()

    plsc.subcore_barrier()

    pltpu.sync_copy(acc_sh.at[pl.ds(s * RPT, RPT)], p_out.at[c, pl.ds(s * RPT, RPT)])


def _tc_dense(x, p, dp, W_self, W_neigh, b, relu):
    BR = NP // 4

    def body(x_ref, p0_ref, p1_ref, d0_ref, d1_ref, ws_ref, wn_ref, b_ref, o_ref):
        deg = d0_ref[0][:, 0:1] + d1_ref[0][:, 0:1]
        recip = 1.0 / jnp.maximum(deg, 1.0)
        hn = (p0_ref[0] + p1_ref[0]) * recip
        acc = jnp.dot(x_ref[...], ws_ref[...], preferred_element_type=jnp.float32)
        acc = acc + jnp.dot(hn, wn_ref[...], preferred_element_type=jnp.float32)
        acc = acc + b_ref[...]
        if relu:
            acc = jnp.maximum(acc, 0.0)
        o_ref[...] = acc

    return pl.pallas_call(
        body,
        grid=(NP // BR,),
        in_specs=[
            pl.BlockSpec((BR, D), lambda i: (i, 0)),
            pl.BlockSpec((1, BR, D), lambda i: (0, i, 0)),
            pl.BlockSpec((1, BR, D), lambda i: (1, i, 0)),
            pl.BlockSpec((1, BR, D), lambda i: (0, i, 0)),
            pl.BlockSpec((1, BR, D), lambda i: (1, i, 0)),
            pl.BlockSpec((D, D), lambda i: (0, 0)),
            pl.BlockSpec((D, D), lambda i: (0, 0)),
            pl.BlockSpec((1, D), lambda i: (0, 0)),
        ],
        out_specs=pl.BlockSpec((BR, D), lambda i: (i, 0)),
        out_shape=jax.ShapeDtypeStruct((N, D), jnp.float32),
    )(x, p, p, dp, dp, W_self, W_neigh, b.reshape(1, D))


def kernel(features, edge_index, W_self1, W_neigh1, b1, W_self2, W_neigh2, b2):
    src = edge_index[0].reshape(NW, EW)
    dst = edge_index[1].reshape(NW, EW)
    npad = EPW - EW
    pad_src = jnp.broadcast_to((jnp.arange(npad, dtype=jnp.int32) * 89) % N, (NW, npad))
    pad_dst = jnp.broadcast_to(N + (jnp.arange(npad, dtype=jnp.int32) % (NP - N)), (NW, npad))
    src_b = jnp.concatenate([src, pad_src], axis=1).reshape(NW, NCH, CH)
    dst_b = jnp.concatenate([dst, pad_dst], axis=1).reshape(NW, NCH, CH)
    dst_g = dst_b.reshape(NW, NG, G, CH)

    z128 = jnp.zeros((NP, D), jnp.float32)
    ones128 = jnp.ones((CH, D), jnp.float32)

    dp = _sc_deg(dst_b, z128, ones128)
    p = _sc_agg(features, src_b, dst_g, z128)
    h = _tc_dense(features, p, dp, W_self1, W_neigh1, b1, relu=True)
    p2 = _sc_agg(h, src_b, dst_g, z128)
    return _tc_dense(h, p2, dp, W_self2, W_neigh2, b2, relu=False)


# speedup vs baseline: 1.0025x; 1.0025x over previous
import functools

import jax
import jax.numpy as jnp
from jax import lax
from jax.experimental import pallas as pl
from jax.experimental.pallas import tpu as pltpu
from jax.experimental.pallas import tpu_sc as plsc

N = 10000
NP = 10112
E = 320000
D = 128
NC = 2
NS = 16
NW = NC * NS
CH = 128
NCH = 80
G = 4
NG = NCH // G
NIT = NCH // 8
EPW = CH * NCH
EW = E // NW
RPT = NP // NS

_mesh = plsc.VectorSubcoreMesh(core_axis_name="c", subcore_axis_name="s")


@functools.partial(
    pl.kernel,
    mesh=_mesh,
    out_type=jax.ShapeDtypeStruct((NC, NP, D), jnp.float32),
    scratch_types=[
        pltpu.VMEM((NCH, CH), jnp.int32),
        pltpu.VMEM((CH, D), jnp.float32),
        pltpu.VMEM_SHARED((NP, D), jnp.float32),
        pltpu.SemaphoreType.DMA,
    ],
)
def _sc_deg(dst_hbm, z128_hbm, ones_hbm,
            d_out,
            dst_v, ones_v, deg_sh, sem):
    c = lax.axis_index("c")
    s = lax.axis_index("s")
    w = c * NS + s

    pltpu.sync_copy(dst_hbm.at[w], dst_v)
    pltpu.sync_copy(z128_hbm.at[pl.ds(s * RPT, RPT)], deg_sh.at[pl.ds(s * RPT, RPT)])
    pltpu.sync_copy(ones_hbm, ones_v)

    plsc.subcore_barrier()

    K = 4

    def _chunk(j, carry):
        pltpu.async_copy(ones_v, deg_sh.at[dst_v.at[j]], sem, add=True)

        @pl.when(j >= K)
        def _():
            pltpu.make_async_copy(ones_v, deg_sh.at[dst_v.at[0]], sem).wait()
        return carry
    lax.fori_loop(0, NCH, _chunk, 0)
    for _ in range(K):
        pltpu.make_async_copy(ones_v, deg_sh.at[dst_v.at[0]], sem).wait()

    plsc.subcore_barrier()

    pltpu.sync_copy(deg_sh.at[pl.ds(s * RPT, RPT)], d_out.at[c, pl.ds(s * RPT, RPT)])


@functools.partial(
    pl.kernel,
    mesh=_mesh,
    out_type=jax.ShapeDtypeStruct((NC, NP, D), jnp.float32),
    scratch_types=[
        pltpu.VMEM((NCH, CH), jnp.int32),
        pltpu.VMEM((G, CH), jnp.int32),
        pltpu.VMEM((G, CH), jnp.int32),
        pltpu.VMEM((CH, D), jnp.float32),
        pltpu.VMEM((CH, D), jnp.float32),
        pltpu.VMEM_SHARED((NP, D), jnp.float32),
        pltpu.SemaphoreType.DMA,
        pltpu.SemaphoreType.DMA,
        pltpu.SemaphoreType.DMA,
        pltpu.SemaphoreType.DMA,
        pltpu.SemaphoreType.DMA,
        pltpu.SemaphoreType.DMA,
    ],
)
def _sc_agg(x_hbm, src_hbm, dstg_hbm, z128_hbm,
            p_out,
            src_v, dstg_a, dstg_b, rows_a, rows_b, acc_sh,
            sem_ga, sem_gb, sem_da, sem_db, sem_sa, sem_sb):
    c = lax.axis_index("c")
    s = lax.axis_index("s")
    w = c * NS + s

    pltpu.sync_copy(src_hbm.at[w], src_v)
    pltpu.async_copy(dstg_hbm.at[w, 0], dstg_a, sem_da)
    pltpu.sync_copy(z128_hbm.at[pl.ds(s * RPT, RPT)], acc_sh.at[pl.ds(s * RPT, RPT)])

    plsc.subcore_barrier()

    rows = (rows_a, rows_b)
    sems = (sem_ga, sem_gb)
    ssems = (sem_sa, sem_sb)

    pltpu.async_copy(x_hbm.at[src_v.at[0, pl.ds(0, 64)]], rows_a.at[pl.ds(0, 64)], sem_ga)
    pltpu.async_copy(x_hbm.at[src_v.at[0, pl.ds(64, 64)]], rows_a.at[pl.ds(64, 64)], sem_ga)

    def _iter(i, carry):
        k0 = 8 * i
        pltpu.make_async_copy(dstg_hbm.at[w, 0], dstg_a, sem_da).wait()
        for t in range(8):
            k = k0 + t
            rb, sg, ss = rows[t % 2], sems[t % 2], ssems[t % 2]
            nrb, nsg, nss = rows[(t + 1) % 2], sems[(t + 1) % 2], ssems[(t + 1) % 2]
            pltpu.make_async_copy(x_hbm.at[src_v.at[k]], rb, sg).wait()
            if t == 0:
                @pl.when(i > 0)
                def _():
                    pltpu.make_async_copy(rows_b, acc_sh.at[dstg_a.at[0]], nss).wait()
                pltpu.async_copy(dstg_hbm.at[w, 2 * i + 1], dstg_b, sem_db)
            else:
                pltpu.make_async_copy(nrb, acc_sh.at[dstg_a.at[0]], nss).wait()
            if t < 7:
                pltpu.async_copy(x_hbm.at[src_v.at[k + 1, pl.ds(0, 64)]],
                                 nrb.at[pl.ds(0, 64)], nsg)
                pltpu.async_copy(x_hbm.at[src_v.at[k + 1, pl.ds(64, 64)]],
                                 nrb.at[pl.ds(64, 64)], nsg)
            else:
                @pl.when(i < NIT - 1)
                def _():
                    pltpu.async_copy(x_hbm.at[src_v.at[k + 1, pl.ds(0, 64)]],
                                     nrb.at[pl.ds(0, 64)], nsg)
                    pltpu.async_copy(x_hbm.at[src_v.at[k + 1, pl.ds(64, 64)]],
                                     nrb.at[pl.ds(64, 64)], nsg)
            if t == 4:
                pltpu.make_async_copy(dstg_hbm.at[w, 0], dstg_b, sem_db).wait()
            dref = dstg_a.at[t] if t < 4 else dstg_b.at[t - 4]
            pltpu.async_copy(rb, acc_sh.at[dref], sem=ss, add=True)

        @pl.when(i < NIT - 1)
        def _():
            pltpu.async_copy(dstg_hbm.at[w, 2 * i + 2], dstg_a, sem_da)
        return carry
    lax.fori_loop(0, NIT, _iter, 0)

    pltpu.make_async_copy(rows_b, acc_sh.at[dstg_a.at[0]], sem_sb).wait()

    plsc.subcore_barrier()

    pltpu.sync_copy(acc_sh.at[pl.ds(s * RPT, RPT)], p_out.at[c, pl.ds(s * RPT, RPT)])


def _tc_dense(x, p, dp, W_self, W_neigh, b, relu):
    BR = NP // 4

    def body(x_ref, p0_ref, p1_ref, d0_ref, d1_ref, ws_ref, wn_ref, b_ref, o_ref):
        deg = d0_ref[0][:, 0:1] + d1_ref[0][:, 0:1]
        recip = 1.0 / jnp.maximum(deg, 1.0)
        hn = (p0_ref[0] + p1_ref[0]) * recip
        acc = jnp.dot(x_ref[...], ws_ref[...], preferred_element_type=jnp.float32)
        acc = acc + jnp.dot(hn, wn_ref[...], preferred_element_type=jnp.float32)
        acc = acc + b_ref[...]
        if relu:
            acc = jnp.maximum(acc, 0.0)
        o_ref[...] = acc

    return pl.pallas_call(
        body,
        grid=(NP // BR,),
        in_specs=[
            pl.BlockSpec((BR, D), lambda i: (i, 0)),
            pl.BlockSpec((1, BR, D), lambda i: (0, i, 0)),
            pl.BlockSpec((1, BR, D), lambda i: (1, i, 0)),
            pl.BlockSpec((1, BR, D), lambda i: (0, i, 0)),
            pl.BlockSpec((1, BR, D), lambda i: (1, i, 0)),
            pl.BlockSpec((D, D), lambda i: (0, 0)),
            pl.BlockSpec((D, D), lambda i: (0, 0)),
            pl.BlockSpec((1, D), lambda i: (0, 0)),
        ],
        out_specs=pl.BlockSpec((BR, D), lambda i: (i, 0)),
        out_shape=jax.ShapeDtypeStruct((N, D), jnp.float32),
    )(x, p, p, dp, dp, W_self, W_neigh, b.reshape(1, D))


def kernel(features, edge_index, W_self1, W_neigh1, b1, W_self2, W_neigh2, b2):
    src = edge_index[0].reshape(NW, EW)
    dst = edge_index[1].reshape(NW, EW)
    npad = EPW - EW
    pad_src = jnp.broadcast_to((jnp.arange(npad, dtype=jnp.int32) * 89) % N, (NW, npad))
    pad_dst = jnp.broadcast_to(N + (jnp.arange(npad, dtype=jnp.int32) % (NP - N)), (NW, npad))
    src_b = jnp.concatenate([src, pad_src], axis=1).reshape(NW, NCH, CH)
    dst_b = jnp.concatenate([dst, pad_dst], axis=1).reshape(NW, NCH, CH)
    dst_g = dst_b.reshape(NW, NG, G, CH)

    z128 = jnp.zeros((NP, D), jnp.float32)
    ones128 = jnp.ones((CH, D), jnp.float32)

    dp = _sc_deg(dst_b, z128, ones128)
    p = _sc_agg(features, src_b, dst_g, z128)
    h = _tc_dense(features, p, dp, W_self1, W_neigh1, b1, relu=True)
    p2 = _sc_agg(h, src_b, dst_g, z128)
    return _tc_dense(h, p2, dp, W_self2, W_neigh2, b2, relu=False)


# speedup vs baseline: 1.0034x; 1.0009x over previous
import functools

import jax
import jax.numpy as jnp
from jax import lax
from jax.experimental import pallas as pl
from jax.experimental.pallas import tpu as pltpu
from jax.experimental.pallas import tpu_sc as plsc

N = 10000
NP = 10112
E = 320000
D = 128
NC = 2
NS = 16
NW = NC * NS
CH = 128
NCH = 80
G = 4
NG = NCH // G
NIT = NCH // 8
EPW = CH * NCH
EW = E // NW
RPT = NP // NS

_mesh = plsc.VectorSubcoreMesh(core_axis_name="c", subcore_axis_name="s")


@functools.partial(
    pl.kernel,
    mesh=_mesh,
    out_type=jax.ShapeDtypeStruct((NC, NP, D), jnp.float32),
    scratch_types=[
        pltpu.VMEM((NCH, CH), jnp.int32),
        pltpu.VMEM((CH, D), jnp.float32),
        pltpu.VMEM_SHARED((NP, D), jnp.float32),
        pltpu.SemaphoreType.DMA,
    ],
)
def _sc_deg(dst_hbm, z128_hbm, ones_hbm,
            d_out,
            dst_v, ones_v, deg_sh, sem):
    c = lax.axis_index("c")
    s = lax.axis_index("s")
    w = c * NS + s

    pltpu.sync_copy(dst_hbm.at[w], dst_v)
    pltpu.sync_copy(z128_hbm.at[pl.ds(s * RPT, RPT)], deg_sh.at[pl.ds(s * RPT, RPT)])
    pltpu.sync_copy(ones_hbm, ones_v)

    plsc.subcore_barrier()

    K = 8

    def _chunk(j, carry):
        pltpu.async_copy(ones_v, deg_sh.at[dst_v.at[j]], sem, add=True)

        @pl.when(j >= K)
        def _():
            pltpu.make_async_copy(ones_v, deg_sh.at[dst_v.at[0]], sem).wait()
        return carry
    lax.fori_loop(0, NCH, _chunk, 0)
    for _ in range(K):
        pltpu.make_async_copy(ones_v, deg_sh.at[dst_v.at[0]], sem).wait()

    plsc.subcore_barrier()

    pltpu.sync_copy(deg_sh.at[pl.ds(s * RPT, RPT)], d_out.at[c, pl.ds(s * RPT, RPT)])


@functools.partial(
    pl.kernel,
    mesh=_mesh,
    out_type=jax.ShapeDtypeStruct((NC, NP, D), jnp.float32),
    scratch_types=[
        pltpu.VMEM((NCH, CH), jnp.int32),
        pltpu.VMEM((G, CH), jnp.int32),
        pltpu.VMEM((G, CH), jnp.int32),
        pltpu.VMEM((CH, D), jnp.float32),
        pltpu.VMEM((CH, D), jnp.float32),
        pltpu.VMEM_SHARED((NP, D), jnp.float32),
        pltpu.SemaphoreType.DMA,
        pltpu.SemaphoreType.DMA,
        pltpu.SemaphoreType.DMA,
        pltpu.SemaphoreType.DMA,
        pltpu.SemaphoreType.DMA,
        pltpu.SemaphoreType.DMA,
    ],
)
def _sc_agg(x_hbm, src_hbm, dstg_hbm, z128_hbm,
            p_out,
            src_v, dstg_a, dstg_b, rows_a, rows_b, acc_sh,
            sem_ga, sem_gb, sem_da, sem_db, sem_sa, sem_sb):
    c = lax.axis_index("c")
    s = lax.axis_index("s")
    w = c * NS + s

    pltpu.sync_copy(src_hbm.at[w], src_v)
    pltpu.async_copy(dstg_hbm.at[w, 0], dstg_a, sem_da)
    pltpu.sync_copy(z128_hbm.at[pl.ds(s * RPT, RPT)], acc_sh.at[pl.ds(s * RPT, RPT)])

    plsc.subcore_barrier()

    rows = (rows_a, rows_b)
    sems = (sem_ga, sem_gb)
    ssems = (sem_sa, sem_sb)

    pltpu.async_copy(x_hbm.at[src_v.at[0, pl.ds(0, 64)]], rows_a.at[pl.ds(0, 64)], sem_ga)
    pltpu.async_copy(x_hbm.at[src_v.at[0, pl.ds(64, 64)]], rows_a.at[pl.ds(64, 64)], sem_ga)

    def _iter(i, carry):
        k0 = 8 * i
        pltpu.make_async_copy(dstg_hbm.at[w, 0], dstg_a, sem_da).wait()
        for t in range(8):
            k = k0 + t
            rb, sg, ss = rows[t % 2], sems[t % 2], ssems[t % 2]
            nrb, nsg, nss = rows[(t + 1) % 2], sems[(t + 1) % 2], ssems[(t + 1) % 2]
            pltpu.make_async_copy(x_hbm.at[src_v.at[k]], rb, sg).wait()
            if t == 0:
                @pl.when(i > 0)
                def _():
                    pltpu.make_async_copy(rows_b, acc_sh.at[dstg_a.at[0]], nss).wait()
                pltpu.async_copy(dstg_hbm.at[w, 2 * i + 1], dstg_b, sem_db)
            else:
                pltpu.make_async_copy(nrb, acc_sh.at[dstg_a.at[0]], nss).wait()
            if t < 7:
                pltpu.async_copy(x_hbm.at[src_v.at[k + 1, pl.ds(0, 64)]],
                                 nrb.at[pl.ds(0, 64)], nsg)
                pltpu.async_copy(x_hbm.at[src_v.at[k + 1, pl.ds(64, 64)]],
                                 nrb.at[pl.ds(64, 64)], nsg)
            else:
                @pl.when(i < NIT - 1)
                def _():
                    pltpu.async_copy(x_hbm.at[src_v.at[k + 1, pl.ds(0, 64)]],
                                     nrb.at[pl.ds(0, 64)], nsg)
                    pltpu.async_copy(x_hbm.at[src_v.at[k + 1, pl.ds(64, 64)]],
                                     nrb.at[pl.ds(64, 64)], nsg)
            if t == 4:
                pltpu.make_async_copy(dstg_hbm.at[w, 0], dstg_b, sem_db).wait()
            dref = dstg_a.at[t] if t < 4 else dstg_b.at[t - 4]
            pltpu.async_copy(rb, acc_sh.at[dref], sem=ss, add=True)

        @pl.when(i < NIT - 1)
        def _():
            pltpu.async_copy(dstg_hbm.at[w, 2 * i + 2], dstg_a, sem_da)
        return carry
    lax.fori_loop(0, NIT, _iter, 0)

    pltpu.make_async_copy(rows_b, acc_sh.at[dstg_a.at[0]], sem_sb).wait()

    plsc.subcore_barrier()

    pltpu.sync_copy(acc_sh.at[pl.ds(s * RPT, RPT)], p_out.at[c, pl.ds(s * RPT, RPT)])


def _tc_dense(x, p, dp, W_self, W_neigh, b, relu):
    BR = NP // 4

    def body(x_ref, p0_ref, p1_ref, d0_ref, d1_ref, ws_ref, wn_ref, b_ref, o_ref):
        deg = d0_ref[0][:, 0:1] + d1_ref[0][:, 0:1]
        recip = 1.0 / jnp.maximum(deg, 1.0)
        hn = (p0_ref[0] + p1_ref[0]) * recip
        acc = jnp.dot(x_ref[...], ws_ref[...], preferred_element_type=jnp.float32)
        acc = acc + jnp.dot(hn, wn_ref[...], preferred_element_type=jnp.float32)
        acc = acc + b_ref[...]
        if relu:
            acc = jnp.maximum(acc, 0.0)
        o_ref[...] = acc

    return pl.pallas_call(
        body,
        grid=(NP // BR,),
        in_specs=[
            pl.BlockSpec((BR, D), lambda i: (i, 0)),
            pl.BlockSpec((1, BR, D), lambda i: (0, i, 0)),
            pl.BlockSpec((1, BR, D), lambda i: (1, i, 0)),
            pl.BlockSpec((1, BR, D), lambda i: (0, i, 0)),
            pl.BlockSpec((1, BR, D), lambda i: (1, i, 0)),
            pl.BlockSpec((D, D), lambda i: (0, 0)),
            pl.BlockSpec((D, D), lambda i: (0, 0)),
            pl.BlockSpec((1, D), lambda i: (0, 0)),
        ],
        out_specs=pl.BlockSpec((BR, D), lambda i: (i, 0)),
        out_shape=jax.ShapeDtypeStruct((N, D), jnp.float32),
    )(x, p, p, dp, dp, W_self, W_neigh, b.reshape(1, D))


def kernel(features, edge_index, W_self1, W_neigh1, b1, W_self2, W_neigh2, b2):
    src = edge_index[0].reshape(NW, EW)
    dst = edge_index[1].reshape(NW, EW)
    npad = EPW - EW
    pad_src = jnp.broadcast_to((jnp.arange(npad, dtype=jnp.int32) * 89) % N, (NW, npad))
    pad_dst = jnp.broadcast_to(N + (jnp.arange(npad, dtype=jnp.int32) % (NP - N)), (NW, npad))
    src_b = jnp.concatenate([src, pad_src], axis=1).reshape(NW, NCH, CH)
    dst_b = jnp.concatenate([dst, pad_dst], axis=1).reshape(NW, NCH, CH)
    dst_g = dst_b.reshape(NW, NG, G, CH)

    z128 = jnp.zeros((NP, D), jnp.float32)
    ones128 = jnp.ones((CH, D), jnp.float32)

    dp = _sc_deg(dst_b, z128, ones128)
    p = _sc_agg(features, src_b, dst_g, z128)
    h = _tc_dense(features, p, dp, W_self1, W_neigh1, b1, relu=True)
    p2 = _sc_agg(h, src_b, dst_g, z128)
    return _tc_dense(h, p2, dp, W_self2, W_neigh2, b2, relu=False)
